# baseline (speedup 1.0000x reference)
import functools

import jax
import jax.numpy as jnp
from jax import lax
from jax.experimental import pallas as pl

N = 8192
IN_CHS = 4096
RED = 1024
K = 2048
M_BLK = 256
GRID_M = N // M_BLK


def _scores_body(x_ref, w1_ref, b1_ref, w2_ref, b2_ref, g_ref, s_ref):
    i = pl.program_id(0)
    xb = x_ref[...]
    h = lax.dot_general(
        xb, w1_ref[...], (((1,), (1,)), ((), ())),
        preferred_element_type=jnp.float32,
    )
    h = jnp.maximum(h + b1_ref[...], 0.0)
    z = jnp.dot(h, w2_ref[...], preferred_element_type=jnp.float32)
    z = z + b2_ref[0, 0]
    a = 1.0 / (1.0 + jnp.exp(-z))
    s_ref[pl.ds(i * M_BLK, M_BLK), :] = a + g_ref[pl.ds(i * M_BLK, M_BLK), :]


def _mask_body(s_ref, m_ref):
    s = s_ref[...]
    u = lax.bitcast_convert_type(s, jnp.uint32)
    flip = jnp.where(
        u >= jnp.uint32(0x80000000),
        jnp.uint32(0xFFFFFFFF),
        jnp.uint32(0x80000000),
    )
    key = u ^ flip

    def bs(_, carry):
        lo, hi = carry
        d = hi - lo
        mid = lo + (d >> jnp.uint32(1)) + (d & jnp.uint32(1))
        cnt = jnp.sum((key >= mid).astype(jnp.int32))
        ok = cnt >= K
        return (jnp.where(ok, mid, lo), jnp.where(ok, hi, mid - jnp.uint32(1)))

    t, _ = lax.fori_loop(
        0, 32, bs, (jnp.uint32(0), jnp.uint32(0xFFFFFFFF))
    )

    gt = key > t
    eq = key == t
    need = (K - jnp.sum(gt.astype(jnp.int32))).astype(jnp.float32)

    eq_f = eq.astype(jnp.float32)
    r0 = lax.broadcasted_iota(jnp.int32, (128, 128), 0)
    c0 = lax.broadcasted_iota(jnp.int32, (128, 128), 1)
    upper_incl = (r0 <= c0).astype(jnp.float32)
    within = jnp.dot(eq_f, upper_incl, preferred_element_type=jnp.float32)
    row_tot = jnp.sum(eq_f, axis=1, keepdims=True)
    r1 = lax.broadcasted_iota(jnp.int32, (64, 64), 0)
    c1 = lax.broadcasted_iota(jnp.int32, (64, 64), 1)
    strict_lower = (c1 < r1).astype(jnp.float32)
    row_pref = jnp.dot(strict_lower, row_tot, preferred_element_type=jnp.float32)
    rank_incl = within + row_pref

    sel = jnp.logical_or(gt, jnp.logical_and(eq, rank_incl <= need))
    m_ref[...] = sel.astype(jnp.float32)


def _apply_body(x_ref, m_ref, o_ref):
    i = pl.program_id(0)
    o_ref[...] = x_ref[...] * m_ref[pl.ds(i * M_BLK, M_BLK), :]


@jax.jit
def kernel(x, W1, b1, W2, b2, gumbels):
    b1r = b1.reshape(1, RED)
    w2c = W2.reshape(RED, 1)
    b2r = b2.reshape(1, 1)
    g2 = gumbels.reshape(N, 1)

    scores = pl.pallas_call(
        _scores_body,
        grid=(GRID_M,),
        in_specs=[
            pl.BlockSpec((M_BLK, IN_CHS), lambda i: (i, 0)),
            pl.BlockSpec((RED, IN_CHS), lambda i: (0, 0)),
            pl.BlockSpec((1, RED), lambda i: (0, 0)),
            pl.BlockSpec((RED, 1), lambda i: (0, 0)),
            pl.BlockSpec((1, 1), lambda i: (0, 0)),
            pl.BlockSpec((N, 1), lambda i: (0, 0)),
        ],
        out_specs=pl.BlockSpec((N, 1), lambda i: (0, 0)),
        out_shape=jax.ShapeDtypeStruct((N, 1), jnp.float32),
    )(x, W1, b1r, w2c, b2r, g2)

    mask = pl.pallas_call(
        _mask_body,
        out_shape=jax.ShapeDtypeStruct((64, 128), jnp.float32),
    )(scores.reshape(64, 128))

    out = pl.pallas_call(
        _apply_body,
        grid=(GRID_M,),
        in_specs=[
            pl.BlockSpec((M_BLK, IN_CHS), lambda i: (i, 0)),
            pl.BlockSpec((N, 1), lambda i: (0, 0)),
        ],
        out_specs=pl.BlockSpec((M_BLK, IN_CHS), lambda i: (i, 0)),
        out_shape=jax.ShapeDtypeStruct((N, IN_CHS), jnp.float32),
    )(x, mask.reshape(N, 1))
    return out
